# baseline (speedup 1.0000x reference)
import functools

import jax
import jax.numpy as jnp
from jax import lax
from jax.experimental import pallas as pl
from jax.experimental.pallas import tpu as pltpu
from jax.experimental.pallas import tpu_sc as plsc

N = 10000
E = 320000
C = 128

NC = 2
NS = 16
NW = NC * NS
EPW = E // NW
CHUNK = 80
NCH = EPW // CHUNK
NP = 10240
ROWS_PT = NP // NS
DPT = NP // NS
NSEC = 5
SCH = NCH // NSEC


def _matmul_body(x_ref, w_ref, o_ref):
    o_ref[...] = jnp.dot(x_ref[...], w_ref[...],
                         preferred_element_type=jnp.float32)


def _tc_matmul(x, w):
    bn = 2000
    return pl.pallas_call(
        _matmul_body,
        grid=(N // bn,),
        in_specs=[
            pl.BlockSpec((bn, C), lambda i: (i, 0)),
            pl.BlockSpec((C, C), lambda i: (0, 0)),
        ],
        out_specs=pl.BlockSpec((bn, C), lambda i: (i, 0)),
        out_shape=jax.ShapeDtypeStruct((N, C), jnp.float32),
    )(x, w)


def _sc_body(m_hbm, src_hbm, dst_hbm, out_sum, out_deg,
             acc_sh, deg_sh, src_v, dst_v, rows0_v, rows1_v, ones_v, zdeg_v,
             gsem0, gsem1, ssem0, ssem1, degsem):
    cid = lax.axis_index("c")
    sid = lax.axis_index("s")
    wid = cid * NS + sid
    rows = (rows0_v, rows1_v)
    gsems = (gsem0, gsem1)
    ssems = (ssem0, ssem1)

    zf16 = jnp.zeros((16,), jnp.float32)

    def _zrow(i, _):
        for k in range(8):
            rows0_v[i, pl.ds(k * 16, 16)] = zf16
        return 0
    lax.fori_loop(0, CHUNK, _zrow, 0)

    def _zdeg(i, _):
        zdeg_v[pl.ds(i * 16, 16)] = zf16
        return 0
    lax.fori_loop(0, DPT // 16, _zdeg, 0)

    for k in range(CHUNK // 16):
        ones_v[pl.ds(k * 16, 16)] = jnp.ones((16,), jnp.float32)

    for b in range(ROWS_PT // CHUNK):
        pltpu.sync_copy(rows0_v,
                        acc_sh.at[pl.ds(sid * ROWS_PT + b * CHUNK, CHUNK)])
    pltpu.sync_copy(zdeg_v, deg_sh.at[pl.ds(sid * DPT, DPT)])

    plsc.subcore_barrier()

    def _sec(s, _):
        pltpu.sync_copy(src_hbm.at[wid, s], src_v)
        pltpu.sync_copy(dst_hbm.at[wid, s], dst_v)
        gdesc = [None, None]
        sdesc = [None]
        ddesc = [None]
        gdesc[0] = pltpu.async_copy(m_hbm.at[src_v.at[0]], rows[0], gsems[0])
        for t in range(SCH):
            b = t % 2
            if sdesc[0] is not None:
                sdesc[0].wait()
            if t + 1 < SCH:
                gdesc[1 - b] = pltpu.async_copy(m_hbm.at[src_v.at[t + 1]],
                                                rows[1 - b], gsems[1 - b])
            gdesc[b].wait()
            sdesc[0] = pltpu.async_copy(rows[b], acc_sh.at[dst_v.at[t]],
                                        ssems[0], add=True)
        sdesc[0].wait()
        for t in range(SCH):
            if ddesc[0] is not None:
                ddesc[0].wait()
            ddesc[0] = pltpu.async_copy(ones_v, deg_sh.at[dst_v.at[t]],
                                        degsem)
        ddesc[0].wait()
        ddesc[0] = None
        return 0
    lax.fori_loop(0, NSEC, _sec, 0)

    plsc.subcore_barrier()

    pltpu.sync_copy(acc_sh.at[pl.ds(sid * ROWS_PT, ROWS_PT)],
                    out_sum.at[cid, pl.ds(sid * ROWS_PT, ROWS_PT)])
    pltpu.sync_copy(deg_sh.at[pl.ds(sid * DPT, DPT)],
                    out_deg.at[cid, pl.ds(sid * DPT, DPT)])


@functools.cache
def _sc_scatter():
    return pl.kernel(
        _sc_body,
        out_type=[
            jax.ShapeDtypeStruct((NC, NP, C), jnp.float32),
            jax.ShapeDtypeStruct((NC, NP), jnp.float32),
        ],
        mesh=plsc.VectorSubcoreMesh(core_axis_name="c", subcore_axis_name="s",
                                    num_cores=NC, num_subcores=NS),
        scratch_types=[
        pltpu.VMEM_SHARED((NP, C), jnp.float32),
        pltpu.VMEM_SHARED((NP,), jnp.float32),
        pltpu.VMEM((SCH, CHUNK), jnp.int32),
        pltpu.VMEM((SCH, CHUNK), jnp.int32),
        pltpu.VMEM((CHUNK, C), jnp.float32),
        pltpu.VMEM((CHUNK, C), jnp.float32),
        pltpu.VMEM((CHUNK,), jnp.float32),
            pltpu.VMEM((DPT,), jnp.float32),
            pltpu.SemaphoreType.DMA,
            pltpu.SemaphoreType.DMA,
            pltpu.SemaphoreType.DMA,
            pltpu.SemaphoreType.DMA,
            pltpu.SemaphoreType.DMA,
        ],
    )


def _gates_body(x_ref, sum_ref, deg_ref, wih_ref, whh_ref, bih_ref, bhh_ref,
                lw_ref, lb_ref, o_ref):
    x = x_ref[...]
    s = sum_ref[0] + sum_ref[1]
    d = deg_ref[0] + deg_ref[1]
    agg = s / jnp.maximum(d, 1.0)
    gi = jnp.dot(agg, wih_ref[...], preferred_element_type=jnp.float32) \
        + bih_ref[...]
    gh = jnp.dot(x, whh_ref[...], preferred_element_type=jnp.float32) \
        + bhh_ref[...]
    r = jax.nn.sigmoid(gi[:, 0:C] + gh[:, 0:C])
    z = jax.nn.sigmoid(gi[:, C:2 * C] + gh[:, C:2 * C])
    n = jnp.tanh(gi[:, 2 * C:] + r * gh[:, 2 * C:])
    h = (1.0 - z) * n + z * x
    gates = jnp.dot(h, lw_ref[...], preferred_element_type=jnp.float32) \
        + lb_ref[...]
    i_g = jax.nn.sigmoid(gates[:, 0:C])
    gg = jnp.tanh(gates[:, C:2 * C])
    o_g = jax.nn.sigmoid(gates[:, 2 * C:])
    o_ref[...] = jax.nn.relu(o_g * jnp.tanh(i_g * gg))


def _tc_gates(x, sum_parts, deg_parts, wih_t, whh_t, b_ih, b_hh, lw_t, lb):
    bn = 2000
    return pl.pallas_call(
        _gates_body,
        grid=(N // bn,),
        in_specs=[
            pl.BlockSpec((bn, C), lambda i: (i, 0)),
            pl.BlockSpec((NC, bn, C), lambda i: (0, i, 0)),
            pl.BlockSpec((NC, bn, 1), lambda i: (0, i, 0)),
            pl.BlockSpec((C, 3 * C), lambda i: (0, 0)),
            pl.BlockSpec((C, 3 * C), lambda i: (0, 0)),
            pl.BlockSpec((1, 3 * C), lambda i: (0, 0)),
            pl.BlockSpec((1, 3 * C), lambda i: (0, 0)),
            pl.BlockSpec((C, 3 * C), lambda i: (0, 0)),
            pl.BlockSpec((1, 3 * C), lambda i: (0, 0)),
        ],
        out_specs=pl.BlockSpec((bn, C), lambda i: (i, 0)),
        out_shape=jax.ShapeDtypeStruct((N, C), jnp.float32),
    )(x, sum_parts, deg_parts, wih_t, whh_t, b_ih, b_hh, lw_t, lb)


def kernel(x, edge_index, W_conv, gru_w_ih, gru_w_hh, gru_b_ih, gru_b_hh,
           lstm_w_ih, lstm_w_hh, lstm_b_ih, lstm_b_hh):
    src = edge_index[0].reshape(NW, NSEC, SCH, CHUNK)
    dst = edge_index[1].reshape(NW, NSEC, SCH, CHUNK)

    m = _tc_matmul(x, W_conv)
    sum_parts, deg_parts = _sc_scatter()(m, src, dst)
    deg_parts = deg_parts[:, :, None]

    lw_sel = jnp.concatenate([lstm_w_ih[0:C], lstm_w_ih[2 * C:]], axis=0)
    lb_sel = (lstm_b_ih + lstm_b_hh)
    lb_sel = jnp.concatenate([lb_sel[0:C], lb_sel[2 * C:]], axis=0)

    return _tc_gates(x, sum_parts, deg_parts,
                     gru_w_ih.T, gru_w_hh.T,
                     gru_b_ih[None, :], gru_b_hh[None, :],
                     lw_sel.T, lb_sel[None, :])


# speedup vs baseline: 1.1403x; 1.1403x over previous
import functools

import jax
import jax.numpy as jnp
from jax import lax
from jax.experimental import pallas as pl
from jax.experimental.pallas import tpu as pltpu
from jax.experimental.pallas import tpu_sc as plsc

N = 10000
E = 320000
C = 128

NC = 2
NS = 16
NW = NC * NS
EPW = E // NW
CHUNK = 80
NCH = EPW // CHUNK
NP = 10240
ROWS_PT = NP // NS
DPT = NP // NS
NSEC = 5
SCH = NCH // NSEC


def _matmul_body(x_ref, w_ref, o_ref):
    o_ref[...] = jnp.dot(x_ref[...], w_ref[...],
                         preferred_element_type=jnp.float32)


def _tc_matmul(x, w):
    bn = 2000
    return pl.pallas_call(
        _matmul_body,
        grid=(N // bn,),
        in_specs=[
            pl.BlockSpec((bn, C), lambda i: (i, 0)),
            pl.BlockSpec((C, C), lambda i: (0, 0)),
        ],
        out_specs=pl.BlockSpec((bn, C), lambda i: (i, 0)),
        out_shape=jax.ShapeDtypeStruct((N, C), jnp.float32),
    )(x, w)


def _sc_body(m_hbm, e_hbm, out_sum, out_deg,
             acc_sh, deg_sh, src_v, dst_v, rows0_v, rows1_v, ones_v, zdeg_v,
             gsem0, gsem1, ssem0, ssem1, degsem):
    cid = lax.axis_index("c")
    sid = lax.axis_index("s")
    wid = cid * NS + sid
    rows = (rows0_v, rows1_v)
    gsems = (gsem0, gsem1)
    ssems = (ssem0, ssem1)

    zf16 = jnp.zeros((16,), jnp.float32)

    def _zrow(i, _):
        for k in range(8):
            rows0_v[i, pl.ds(k * 16, 16)] = zf16
        return 0
    lax.fori_loop(0, CHUNK, _zrow, 0)

    def _zdeg(i, _):
        zdeg_v[pl.ds(i * 16, 16)] = zf16
        return 0
    lax.fori_loop(0, DPT // 16, _zdeg, 0)

    for k in range(CHUNK // 16):
        ones_v[pl.ds(k * 16, 16)] = jnp.ones((16,), jnp.float32)

    for b in range(ROWS_PT // CHUNK):
        pltpu.sync_copy(rows0_v,
                        acc_sh.at[pl.ds(sid * ROWS_PT + b * CHUNK, CHUNK)])
    pltpu.sync_copy(zdeg_v, deg_sh.at[pl.ds(sid * DPT, DPT)])

    plsc.subcore_barrier()

    def _sec(s, _):
        pltpu.sync_copy(e_hbm.at[wid, s, 0], src_v)
        pltpu.sync_copy(e_hbm.at[wid, s, 1], dst_v)
        gdesc = [None, None]
        sdesc = [None]
        gdesc[0] = pltpu.async_copy(m_hbm.at[src_v.at[0]], rows[0], gsems[0])
        for t in range(SCH):
            b = t % 2
            if sdesc[0] is not None:
                sdesc[0].wait()
            if t + 1 < SCH:
                gdesc[1 - b] = pltpu.async_copy(m_hbm.at[src_v.at[t + 1]],
                                                rows[1 - b], gsems[1 - b])
            gdesc[b].wait()
            sdesc[0] = pltpu.async_copy(rows[b], acc_sh.at[dst_v.at[t]],
                                        ssems[0], add=True)
            pltpu.sync_copy(ones_v, deg_sh.at[dst_v.at[t]], add=True)
        sdesc[0].wait()
        return 0
    lax.fori_loop(0, NSEC, _sec, 0)

    plsc.subcore_barrier()

    pltpu.sync_copy(acc_sh.at[pl.ds(sid * ROWS_PT, ROWS_PT)],
                    out_sum.at[cid, pl.ds(sid * ROWS_PT, ROWS_PT)])
    pltpu.sync_copy(deg_sh.at[pl.ds(sid * DPT, DPT)],
                    out_deg.at[cid, pl.ds(sid * DPT, DPT)])


@functools.cache
def _sc_scatter():
    return pl.kernel(
        _sc_body,
        out_type=[
            jax.ShapeDtypeStruct((NC, NP, C), jnp.float32),
            jax.ShapeDtypeStruct((NC, NP), jnp.float32),
        ],
        mesh=plsc.VectorSubcoreMesh(core_axis_name="c", subcore_axis_name="s",
                                    num_cores=NC, num_subcores=NS),
        scratch_types=[
        pltpu.VMEM_SHARED((NP, C), jnp.float32),
        pltpu.VMEM_SHARED((NP,), jnp.float32),
        pltpu.VMEM((SCH, CHUNK), jnp.int32),
        pltpu.VMEM((SCH, CHUNK), jnp.int32),
        pltpu.VMEM((CHUNK, C), jnp.float32),
        pltpu.VMEM((CHUNK, C), jnp.float32),
        pltpu.VMEM((CHUNK,), jnp.float32),
            pltpu.VMEM((DPT,), jnp.float32),
            pltpu.SemaphoreType.DMA,
            pltpu.SemaphoreType.DMA,
            pltpu.SemaphoreType.DMA,
            pltpu.SemaphoreType.DMA,
            pltpu.SemaphoreType.DMA,
        ],
    )


def _gates_body(x_ref, sum_ref, deg_ref, wih_ref, whh_ref, bih_ref, bhh_ref,
                lw_ref, lbi_ref, lbh_ref, o_ref):
    cdims = (((1,), (1,)), ((), ()))
    x = x_ref[...]
    s = sum_ref[0] + sum_ref[1]
    d = lax.dot_general(deg_ref[...], jnp.ones((NC, 1), jnp.float32),
                        (((0,), (0,)), ((), ())),
                        preferred_element_type=jnp.float32)
    agg = s / jnp.maximum(d, 1.0)
    gi = lax.dot_general(agg, wih_ref[...], cdims,
                         preferred_element_type=jnp.float32) + bih_ref[...]
    gh = lax.dot_general(x, whh_ref[...], cdims,
                         preferred_element_type=jnp.float32) + bhh_ref[...]
    r = jax.nn.sigmoid(gi[:, 0:C] + gh[:, 0:C])
    z = jax.nn.sigmoid(gi[:, C:2 * C] + gh[:, C:2 * C])
    n = jnp.tanh(gi[:, 2 * C:] + r * gh[:, 2 * C:])
    h = (1.0 - z) * n + z * x
    lb = lbi_ref[...] + lbh_ref[...]
    gates = lax.dot_general(h, lw_ref[...], cdims,
                            preferred_element_type=jnp.float32) + lb
    i_g = jax.nn.sigmoid(gates[:, 0:C])
    gg = jnp.tanh(gates[:, 2 * C:3 * C])
    o_g = jax.nn.sigmoid(gates[:, 3 * C:])
    o_ref[...] = jax.nn.relu(o_g * jnp.tanh(i_g * gg))


def _tc_gates(x, sum_parts, deg_parts, w_ih, w_hh, b_ih, b_hh, lw, lbi, lbh):
    bn = 5120
    return pl.pallas_call(
        _gates_body,
        grid=(2,),
        in_specs=[
            pl.BlockSpec((bn, C), lambda i: (i, 0)),
            pl.BlockSpec((NC, bn, C), lambda i: (0, i, 0)),
            pl.BlockSpec((NC, bn), lambda i: (0, i)),
            pl.BlockSpec((3 * C, C), lambda i: (0, 0)),
            pl.BlockSpec((3 * C, C), lambda i: (0, 0)),
            pl.BlockSpec((3 * C,), lambda i: (0,)),
            pl.BlockSpec((3 * C,), lambda i: (0,)),
            pl.BlockSpec((4 * C, C), lambda i: (0, 0)),
            pl.BlockSpec((4 * C,), lambda i: (0,)),
            pl.BlockSpec((4 * C,), lambda i: (0,)),
        ],
        out_specs=pl.BlockSpec((bn, C), lambda i: (i, 0)),
        out_shape=jax.ShapeDtypeStruct((N, C), jnp.float32),
    )(x, sum_parts, deg_parts, w_ih, w_hh, b_ih, b_hh, lw, lbi, lbh)


def kernel(x, edge_index, W_conv, gru_w_ih, gru_w_hh, gru_b_ih, gru_b_hh,
           lstm_w_ih, lstm_w_hh, lstm_b_ih, lstm_b_hh):
    edges = edge_index.reshape(2, NW, NSEC, SCH, CHUNK).transpose(1, 2, 0, 3, 4)

    m = _tc_matmul(x, W_conv)
    sum_parts, deg_parts = _sc_scatter()(m, edges)

    return _tc_gates(x, sum_parts, deg_parts,
                     gru_w_ih, gru_w_hh, gru_b_ih, gru_b_hh,
                     lstm_w_ih, lstm_b_ih, lstm_b_hh)
